# baseline (speedup 1.0000x reference)
import functools

import jax
import jax.numpy as jnp
from jax import lax
from jax.experimental import pallas as pl
from jax.experimental.pallas import tpu as pltpu

_EPS = 1e-5


def _fused_kernel(x_ref, wdw_ref, w2_ref, g1_ref, b1_ref, g2_ref, b2_ref,
                  y_ref,
                  xf_ref, hbuf_ref, st1_ref, a1c1_ref, ms_ref, ss_ref,
                  a2c2_ref,
                  *, bb, cin, ww, hw, padhw, inv_count):
    ph = pl.program_id(0)
    i = pl.program_id(1)

    cch = min(512, hw)
    pch = min(2 * cch, hw)

    @pl.when(ph == 0)
    def _phase0():
        @pl.when(i == 0)
        def _init():
            xf_ref[...] = jnp.zeros_like(xf_ref)
            st1_ref[...] = jnp.zeros_like(st1_ref)

        for b in range(bb):
            xf_ref[b, :, 2 * ww:2 * ww + hw] = x_ref[b].reshape(cin, hw)

        lane = lax.broadcasted_iota(jnp.int32, (1, cch), 1) % ww
        fa = (lane != 0).astype(jnp.float32)
        fb = (lane != ww - 1).astype(jnp.float32)

        st_s = None
        st_q = None
        for b in range(bb):
            for c in range(hw // cch):
                c0 = c * cch
                acc = None
                for kh in range(3):
                    for kw in range(3):
                        s = (ww - 1) + kh * ww + kw + c0
                        t = xf_ref[b, :, s:s + cch]
                        if kw == 0:
                            t = t * fa
                        elif kw == 2:
                            t = t * fb
                        t = wdw_ref[kh * 3 + kw] * t
                        acc = t if acc is None else acc + t
                hbuf_ref[i * bb + b, :, c0:c0 + cch] = acc
                ps = jnp.sum(acc, axis=1, keepdims=True)
                pq = jnp.sum(acc * acc, axis=1, keepdims=True)
                st_s = ps if st_s is None else st_s + ps
                st_q = pq if st_q is None else st_q + pq
        st1_ref[0] += st_s
        st1_ref[1] += st_q

    @pl.when(ph == 1)
    def _phase1():
        @pl.when(i == 0)
        def _fold_bn1():
            mean = st1_ref[0] * inv_count
            var = jnp.maximum(st1_ref[1] * inv_count - mean * mean, 0.0)
            a1 = g1_ref[...] * lax.rsqrt(var + _EPS)
            a1c1_ref[0] = a1
            a1c1_ref[1] = b1_ref[...] - mean * a1
            ms_ref[...] = jnp.zeros_like(ms_ref)
            ss_ref[...] = jnp.zeros_like(ss_ref)

        st_s = None
        for b in range(bb):
            idx = i * bb + b
            for c in range(hw // pch):
                sl = pl.ds(c * pch, pch)
                g = jnp.clip(hbuf_ref[idx, :, sl] * a1c1_ref[0]
                             + a1c1_ref[1], 0.0, 6.0)
                hbuf_ref[idx, :, sl] = g
                ps = jnp.sum(g, axis=1, keepdims=True)
                st_s = ps if st_s is None else st_s + ps
        ss_ref[...] += st_s
        gram = None
        for b in range(bb):
            gb = hbuf_ref[i * bb + b]
            p = lax.dot_general(gb, gb, (((1,), (1,)), ((), ())),
                                preferred_element_type=jnp.float32)
            gram = p if gram is None else gram + p
        ms_ref[...] += gram

    @pl.when(ph == 2)
    def _phase2():
        @pl.when(i == 0)
        def _fold_bn2():
            w2v = w2_ref[...]
            meanz = lax.dot_general(
                w2v, ss_ref[...], (((1,), (0,)), ((), ())),
                preferred_element_type=jnp.float32) * inv_count
            t = jnp.dot(w2v, ms_ref[...],
                        preferred_element_type=jnp.float32)
            ez2 = jnp.sum(t * w2v, axis=1, keepdims=True) * inv_count
            var = jnp.maximum(ez2 - meanz * meanz, 0.0)
            a2 = g2_ref[...] * lax.rsqrt(var + _EPS)
            a2c2_ref[0] = a2
            a2c2_ref[1] = b2_ref[...] - meanz * a2

        for b in range(bb):
            idx = i * bb + b
            for c in range(hw // cch):
                c0 = c * cch
                z = jnp.dot(w2_ref[...], hbuf_ref[idx, :, c0:c0 + cch],
                            preferred_element_type=jnp.float32)
                z = z * a2c2_ref[0] + a2c2_ref[1]
                r0 = c0 // ww
                y_ref[b, :, r0:r0 + cch // ww, :] = z.reshape(
                    z.shape[0], cch // ww, ww)


@jax.jit
def _dsconv(x, w_dw, g1, b1, w_pw, g2, b2):
    n_img, cin, hh, ww = x.shape
    kk = w_dw.shape[-1]
    assert kk == 3 and kk // 2 == 1
    cout = w_pw.shape[0]
    hw = hh * ww
    padhw = hw + 4 * ww
    inv_count = 1.0 / float(n_img * hw)
    bb = 2
    assert n_img % bb == 0
    nsteps = n_img // bb

    x2 = x.astype(jnp.float32)
    wdw9 = w_dw.reshape(cin, kk * kk).T.reshape(kk * kk, cin, 1)
    wdw9 = wdw9.astype(jnp.float32)
    w2 = w_pw.reshape(cout, cin).astype(jnp.float32)
    g1r = g1.reshape(cin, 1).astype(jnp.float32)
    b1r = b1.reshape(cin, 1).astype(jnp.float32)
    g2r = g2.reshape(cout, 1).astype(jnp.float32)
    b2r = b2.reshape(cout, 1).astype(jnp.float32)

    y = pl.pallas_call(
        functools.partial(_fused_kernel, bb=bb, cin=cin, ww=ww, hw=hw,
                          padhw=padhw, inv_count=inv_count),
        grid=(3, nsteps),
        in_specs=[
            pl.BlockSpec((bb, cin, hh, ww),
                         lambda ph, i: (jnp.where(ph == 0, i, 0), 0, 0, 0)),
            pl.BlockSpec((kk * kk, cin, 1), lambda ph, i: (0, 0, 0)),
            pl.BlockSpec((cout, cin), lambda ph, i: (0, 0)),
            pl.BlockSpec((cin, 1), lambda ph, i: (0, 0)),
            pl.BlockSpec((cin, 1), lambda ph, i: (0, 0)),
            pl.BlockSpec((cout, 1), lambda ph, i: (0, 0)),
            pl.BlockSpec((cout, 1), lambda ph, i: (0, 0)),
        ],
        out_specs=pl.BlockSpec(
            (bb, cout, hh, ww),
            lambda ph, i: (jnp.where(ph == 2, i, 0), 0, 0, 0)),
        out_shape=jax.ShapeDtypeStruct((n_img, cout, hh, ww), jnp.float32),
        scratch_shapes=[
            pltpu.VMEM((bb, cin, padhw), jnp.float32),
            pltpu.VMEM((n_img, cin, hw), jnp.float32),
            pltpu.VMEM((2, cin, 1), jnp.float32),
            pltpu.VMEM((2, cin, 1), jnp.float32),
            pltpu.VMEM((cin, cin), jnp.float32),
            pltpu.VMEM((cin, 1), jnp.float32),
            pltpu.VMEM((2, cout, 1), jnp.float32),
        ],
        compiler_params=pltpu.CompilerParams(
            dimension_semantics=("arbitrary", "arbitrary"),
            vmem_limit_bytes=58 * 1024 * 1024,
        ),
        cost_estimate=pl.CostEstimate(
            flops=2 * n_img * cin * hw * kk * kk
            + 2 * n_img * hw * cin * (cin + cout),
            transcendentals=0,
            bytes_accessed=4 * (x2.size + n_img * cout * hw),
        ),
    )(x2, wdw9, w2, g1r, b1r, g2r, b2r)

    return y


def kernel(x, w_dw, g1, b1, w_pw, g2, b2):
    return _dsconv(x, w_dw, g1, b1, w_pw, g2, b2)


# speedup vs baseline: 2.6828x; 2.6828x over previous
import functools

import jax
import jax.numpy as jnp
from jax import lax
from jax.experimental import pallas as pl
from jax.experimental.pallas import tpu as pltpu

_EPS = 1e-5


def _fused_kernel(x_ref, wdw_ref, w2_ref, g1_ref, b1_ref, g2_ref, b2_ref,
                  y_ref,
                  xf_ref, hbuf_ref, st1_ref, a1c1_ref, ms_ref, ss_ref,
                  a2c2_ref,
                  *, bb, cin, ww, hw, padhw, inv_count):
    ph = pl.program_id(0)
    i = pl.program_id(1)

    @pl.when(ph == 0)
    def _phase0():
        @pl.when(i == 0)
        def _init():
            xf_ref[...] = jnp.zeros_like(xf_ref)
            st1_ref[...] = jnp.zeros_like(st1_ref)

        for b in range(bb):
            xf_ref[b, :, 2 * ww:2 * ww + hw] = x_ref[b].reshape(cin, hw)
        xfv = xf_ref[...]
        lane = lax.broadcasted_iota(jnp.int32, (1, hw), 1) % ww
        fa = (lane != 0).astype(jnp.float32)
        fb = (lane != ww - 1).astype(jnp.float32)
        acc = None
        for kh in range(3):
            for kw in range(3):
                s = (ww - 1) + kh * ww + kw
                t = wdw_ref[kh * 3 + kw] * xfv[:, :, s:s + hw]
                if kw == 0:
                    t = t * fa
                elif kw == 2:
                    t = t * fb
                acc = t if acc is None else acc + t
        hbuf_ref[pl.ds(i * bb, bb)] = acc
        st1_ref[0] += jnp.sum(acc, axis=(0, 2), keepdims=True)[0]
        st1_ref[1] += jnp.sum(acc * acc, axis=(0, 2), keepdims=True)[0]

    @pl.when(ph == 1)
    def _phase1():
        @pl.when(i == 0)
        def _fold_bn1():
            mean = st1_ref[0] * inv_count
            var = jnp.maximum(st1_ref[1] * inv_count - mean * mean, 0.0)
            a1 = g1_ref[...] * lax.rsqrt(var + _EPS)
            a1c1_ref[0] = a1
            a1c1_ref[1] = b1_ref[...] - mean * a1
            ms_ref[...] = jnp.zeros_like(ms_ref)
            ss_ref[...] = jnp.zeros_like(ss_ref)

        g = jnp.clip(hbuf_ref[pl.ds(i * bb, bb)] * a1c1_ref[0]
                     + a1c1_ref[1], 0.0, 6.0)
        hbuf_ref[pl.ds(i * bb, bb)] = g
        ss_ref[...] += jnp.sum(g, axis=(0, 2), keepdims=True)[0]
        gram = None
        for b in range(bb):
            gb = hbuf_ref[i * bb + b]
            p = lax.dot_general(gb, gb, (((1,), (1,)), ((), ())),
                                preferred_element_type=jnp.float32)
            gram = p if gram is None else gram + p
        ms_ref[...] += gram

    @pl.when(ph == 2)
    def _phase2():
        @pl.when(i == 0)
        def _fold_bn2():
            w2v = w2_ref[...]
            meanz = lax.dot_general(
                w2v, ss_ref[...], (((1,), (0,)), ((), ())),
                preferred_element_type=jnp.float32) * inv_count
            t = jnp.dot(w2v, ms_ref[...],
                        preferred_element_type=jnp.float32)
            ez2 = jnp.sum(t * w2v, axis=1, keepdims=True) * inv_count
            var = jnp.maximum(ez2 - meanz * meanz, 0.0)
            a2 = g2_ref[...] * lax.rsqrt(var + _EPS)
            a2c2_ref[0] = a2
            a2c2_ref[1] = b2_ref[...] - meanz * a2

        for b in range(bb):
            z = jnp.dot(w2_ref[...], hbuf_ref[i * bb + b],
                        preferred_element_type=jnp.float32)
            z = z * a2c2_ref[0] + a2c2_ref[1]
            y_ref[b] = z.reshape(z.shape[0], hw // ww, ww)


@jax.jit
def _dsconv(x, w_dw, g1, b1, w_pw, g2, b2):
    n_img, cin, hh, ww = x.shape
    kk = w_dw.shape[-1]
    assert kk == 3 and kk // 2 == 1
    cout = w_pw.shape[0]
    hw = hh * ww
    padhw = hw + 4 * ww
    inv_count = 1.0 / float(n_img * hw)
    bb = 2
    assert n_img % bb == 0
    nsteps = n_img // bb

    x2 = x.astype(jnp.float32)
    wdw9 = w_dw.reshape(cin, kk * kk).T.reshape(kk * kk, cin, 1)
    wdw9 = wdw9.astype(jnp.float32)
    w2 = w_pw.reshape(cout, cin).astype(jnp.float32)
    g1r = g1.reshape(cin, 1).astype(jnp.float32)
    b1r = b1.reshape(cin, 1).astype(jnp.float32)
    g2r = g2.reshape(cout, 1).astype(jnp.float32)
    b2r = b2.reshape(cout, 1).astype(jnp.float32)

    y = pl.pallas_call(
        functools.partial(_fused_kernel, bb=bb, cin=cin, ww=ww, hw=hw,
                          padhw=padhw, inv_count=inv_count),
        grid=(3, nsteps),
        in_specs=[
            pl.BlockSpec((bb, cin, hh, ww),
                         lambda ph, i: (jnp.where(ph == 0, i, 0), 0, 0, 0)),
            pl.BlockSpec((kk * kk, cin, 1), lambda ph, i: (0, 0, 0)),
            pl.BlockSpec((cout, cin), lambda ph, i: (0, 0)),
            pl.BlockSpec((cin, 1), lambda ph, i: (0, 0)),
            pl.BlockSpec((cin, 1), lambda ph, i: (0, 0)),
            pl.BlockSpec((cout, 1), lambda ph, i: (0, 0)),
            pl.BlockSpec((cout, 1), lambda ph, i: (0, 0)),
        ],
        out_specs=pl.BlockSpec(
            (bb, cout, hh, ww),
            lambda ph, i: (jnp.where(ph == 2, i, 0), 0, 0, 0)),
        out_shape=jax.ShapeDtypeStruct((n_img, cout, hh, ww), jnp.float32),
        scratch_shapes=[
            pltpu.VMEM((bb, cin, padhw), jnp.float32),
            pltpu.VMEM((n_img, cin, hw), jnp.float32),
            pltpu.VMEM((2, cin, 1), jnp.float32),
            pltpu.VMEM((2, cin, 1), jnp.float32),
            pltpu.VMEM((cin, cin), jnp.float32),
            pltpu.VMEM((cin, 1), jnp.float32),
            pltpu.VMEM((2, cout, 1), jnp.float32),
        ],
        compiler_params=pltpu.CompilerParams(
            dimension_semantics=("arbitrary", "arbitrary"),
            vmem_limit_bytes=58 * 1024 * 1024,
        ),
        cost_estimate=pl.CostEstimate(
            flops=2 * n_img * cin * hw * kk * kk
            + 2 * n_img * hw * cin * (cin + cout),
            transcendentals=0,
            bytes_accessed=4 * (x2.size + n_img * cout * hw),
        ),
    )(x2, wdw9, w2, g1r, b1r, g2r, b2r)

    return y


def kernel(x, w_dw, g1, b1, w_pw, g2, b2):
    return _dsconv(x, w_dw, g1, b1, w_pw, g2, b2)


# speedup vs baseline: 2.6982x; 1.0058x over previous
import functools

import jax
import jax.numpy as jnp
from jax import lax
from jax.experimental import pallas as pl
from jax.experimental.pallas import tpu as pltpu

_EPS = 1e-5


def _dw_kernel(x_ref, wdw_ref, h_ref, st1_ref, xf_ref,
               *, bb, cin, ww, hw):
    i = pl.program_id(1)

    @pl.when(i == 0)
    def _init():
        xf_ref[...] = jnp.zeros_like(xf_ref)
        st1_ref[...] = jnp.zeros_like(st1_ref)

    for b in range(bb):
        xf_ref[b, :, 2 * ww:2 * ww + hw] = x_ref[b].reshape(cin, hw)
    xfv = xf_ref[...]
    lane = lax.broadcasted_iota(jnp.int32, (1, hw), 1) % ww
    fa = (lane != 0).astype(jnp.float32)
    fb = (lane != ww - 1).astype(jnp.float32)
    acc = None
    for kh in range(3):
        for kw in range(3):
            s = (ww - 1) + kh * ww + kw
            t = wdw_ref[kh * 3 + kw] * xfv[:, :, s:s + hw]
            if kw == 0:
                t = t * fa
            elif kw == 2:
                t = t * fb
            acc = t if acc is None else acc + t
    h_ref[...] = acc
    st1_ref[0, 0] += jnp.sum(acc, axis=(0, 2), keepdims=True)[0]
    st1_ref[0, 1] += jnp.sum(acc * acc, axis=(0, 2), keepdims=True)[0]


def _bn1_gram_kernel(h_ref, a1_ref, c1_ref, g_ref, ms_ref, ss_ref,
                     *, bb):
    i = pl.program_id(1)

    @pl.when(i == 0)
    def _init():
        ms_ref[...] = jnp.zeros_like(ms_ref)
        ss_ref[...] = jnp.zeros_like(ss_ref)

    g = jnp.clip(h_ref[...] * a1_ref[...] + c1_ref[...], 0.0, 6.0)
    g_ref[...] = g
    ss_ref[0] += jnp.sum(g, axis=(0, 2), keepdims=True)[0]
    gram = None
    for b in range(bb):
        gb = g_ref[b]
        p = lax.dot_general(gb, gb, (((1,), (1,)), ((), ())),
                            preferred_element_type=jnp.float32)
        gram = p if gram is None else gram + p
    ms_ref[0] += gram


def _pw_kernel(g_ref, w2_ref, a2_ref, c2_ref, y_ref, *, bb, ww, hw):
    for b in range(bb):
        z = jnp.dot(w2_ref[...], g_ref[b],
                    preferred_element_type=jnp.float32)
        z = z * a2_ref[...] + c2_ref[...]
        y_ref[b] = z.reshape(z.shape[0], hw // ww, ww)


@jax.jit
def _dsconv(x, w_dw, g1, b1, w_pw, g2, b2):
    n_img, cin, hh, ww = x.shape
    kk = w_dw.shape[-1]
    assert kk == 3 and kk // 2 == 1
    cout = w_pw.shape[0]
    hw = hh * ww
    padhw = hw + 4 * ww
    count = float(n_img * hw)
    ncore = 2
    bb = max(1, min(4, n_img // ncore))
    assert n_img % (ncore * bb) == 0
    nsteps = n_img // (ncore * bb)

    x2 = x.astype(jnp.float32)
    wdw9 = w_dw.reshape(cin, kk * kk).T.reshape(kk * kk, cin, 1)
    wdw9 = wdw9.astype(jnp.float32)
    w2 = w_pw.reshape(cout, cin).astype(jnp.float32)

    h, st1 = pl.pallas_call(
        functools.partial(_dw_kernel, bb=bb, cin=cin, ww=ww, hw=hw),
        grid=(ncore, nsteps),
        in_specs=[
            pl.BlockSpec((bb, cin, hh, ww),
                         lambda c, i, ns=nsteps: (c * ns + i, 0, 0, 0)),
            pl.BlockSpec((kk * kk, cin, 1), lambda c, i: (0, 0, 0)),
        ],
        out_specs=[
            pl.BlockSpec((bb, cin, hw),
                         lambda c, i, ns=nsteps: (c * ns + i, 0, 0)),
            pl.BlockSpec((1, 2, cin, 1), lambda c, i: (c, 0, 0, 0)),
        ],
        out_shape=[
            jax.ShapeDtypeStruct((n_img, cin, hw), jnp.float32),
            jax.ShapeDtypeStruct((ncore, 2, cin, 1), jnp.float32),
        ],
        scratch_shapes=[pltpu.VMEM((bb, cin, padhw), jnp.float32)],
        compiler_params=pltpu.CompilerParams(
            dimension_semantics=("parallel", "arbitrary"),
            vmem_limit_bytes=48 * 1024 * 1024,
        ),
    )(x2, wdw9)

    mean1 = jnp.sum(st1[:, 0], axis=0) / count
    var1 = jnp.maximum(jnp.sum(st1[:, 1], axis=0) / count
                       - mean1 * mean1, 0.0)
    a1 = g1.reshape(cin, 1) * lax.rsqrt(var1 + _EPS)
    c1 = b1.reshape(cin, 1) - mean1 * a1

    g, ms, ss = pl.pallas_call(
        functools.partial(_bn1_gram_kernel, bb=bb),
        grid=(ncore, nsteps),
        in_specs=[
            pl.BlockSpec((bb, cin, hw),
                         lambda c, i, ns=nsteps: (c * ns + i, 0, 0)),
            pl.BlockSpec((cin, 1), lambda c, i: (0, 0)),
            pl.BlockSpec((cin, 1), lambda c, i: (0, 0)),
        ],
        out_specs=[
            pl.BlockSpec((bb, cin, hw),
                         lambda c, i, ns=nsteps: (c * ns + i, 0, 0)),
            pl.BlockSpec((1, cin, cin), lambda c, i: (c, 0, 0)),
            pl.BlockSpec((1, cin, 1), lambda c, i: (c, 0, 0)),
        ],
        out_shape=[
            jax.ShapeDtypeStruct((n_img, cin, hw), jnp.float32),
            jax.ShapeDtypeStruct((ncore, cin, cin), jnp.float32),
            jax.ShapeDtypeStruct((ncore, cin, 1), jnp.float32),
        ],
        input_output_aliases={0: 0},
        compiler_params=pltpu.CompilerParams(
            dimension_semantics=("parallel", "arbitrary"),
            vmem_limit_bytes=48 * 1024 * 1024,
        ),
    )(h, a1, c1)

    msum = jnp.sum(ms, axis=0)
    ssum = jnp.sum(ss, axis=0)
    mean2 = jnp.dot(w2, ssum) / count
    ez2 = jnp.sum(jnp.dot(w2, msum) * w2, axis=1,
                  keepdims=True) / count
    var2 = jnp.maximum(ez2 - mean2 * mean2, 0.0)
    a2 = g2.reshape(cout, 1) * lax.rsqrt(var2 + _EPS)
    c2 = b2.reshape(cout, 1) - mean2 * a2

    y = pl.pallas_call(
        functools.partial(_pw_kernel, bb=bb, ww=ww, hw=hw),
        grid=(ncore, nsteps),
        in_specs=[
            pl.BlockSpec((bb, cin, hw),
                         lambda c, i, ns=nsteps: (c * ns + i, 0, 0)),
            pl.BlockSpec((cout, cin), lambda c, i: (0, 0)),
            pl.BlockSpec((cout, 1), lambda c, i: (0, 0)),
            pl.BlockSpec((cout, 1), lambda c, i: (0, 0)),
        ],
        out_specs=pl.BlockSpec(
            (bb, cout, hh, ww),
            lambda c, i, ns=nsteps: (c * ns + i, 0, 0, 0)),
        out_shape=jax.ShapeDtypeStruct((n_img, cout, hh, ww), jnp.float32),
        compiler_params=pltpu.CompilerParams(
            dimension_semantics=("parallel", "parallel"),
            vmem_limit_bytes=48 * 1024 * 1024,
        ),
    )(g, w2, a2, c2)

    return y


def kernel(x, w_dw, g1, b1, w_pw, g2, b2):
    return _dsconv(x, w_dw, g1, b1, w_pw, g2, b2)
